# baseline (speedup 1.0000x reference)
import functools

import jax
import jax.numpy as jnp
from jax import lax
from jax.experimental import pallas as pl
from jax.experimental.pallas import tpu as pltpu
from jax.experimental.pallas import tpu_sc as plsc

_B = 16384
_D = 16
_ODIM = 10
_NIG = 10
_NGG = 21

_NC, _NS, _L = 2, 16, 16
_NW = _NC * _NS
_BPW = _B // _NW
_CW = 128
_CH = _BPW // _CW
_G = _BPW // _L

_mesh = plsc.VectorSubcoreMesh(core_axis_name="c", subcore_axis_name="s")


@functools.partial(
    pl.kernel,
    mesh=_mesh,
    out_type=jax.ShapeDtypeStruct((_B, _ODIM), jnp.float32),
    compiler_params=pltpu.CompilerParams(
        needs_layout_passes=False, use_tc_tiling_on_sc=False),
    scratch_types=[
        pltpu.VMEM((_CH, _CW), jnp.int32),
        pltpu.VMEM((_BPW, _D), jnp.float32),
        pltpu.VMEM((_BPW,), jnp.int32),
        pltpu.VMEM((_BPW,), jnp.int32),
        pltpu.VMEM((_ODIM * _D,), jnp.float32),
        pltpu.VMEM((_NIG * _ODIM,), jnp.float32),
        pltpu.VMEM((_NGG * _ODIM,), jnp.float32),
        pltpu.VMEM((_BPW, _ODIM), jnp.float32),
        pltpu.SemaphoreType.DMA,
    ],
)
def _tower_sc(ids_hbm, ig_hbm, gg_hbm, tbl_hbm, w16_hbm, t1_hbm, t2_hbm,
              out_hbm, idx_v, rows_v, ig_v, gg_v, w16_v, t1_v, t2_v, out_v,
              sem):
    c = lax.axis_index("c")
    s = lax.axis_index("s")
    wid = s * _NC + c
    base = wid * _BPW

    pltpu.sync_copy(ids_hbm.at[wid], idx_v)
    gathers = [
        pltpu.async_copy(tbl_hbm.at[idx_v.at[ci]],
                         rows_v.at[pl.ds(ci * _CW, _CW)], sem)
        for ci in range(_CH)
    ]
    pltpu.sync_copy(ig_hbm.at[wid], ig_v)
    pltpu.sync_copy(gg_hbm.at[wid], gg_v)
    pltpu.sync_copy(w16_hbm, w16_v)
    pltpu.sync_copy(t1_hbm, t1_v)
    pltpu.sync_copy(t2_hbm, t2_v)
    for g in gathers:
        g.wait()

    wrows = [w16_v[pl.ds(j * _D, _D)] for j in range(_ODIM)]
    ws = [[wrows[j][k] for k in range(_D)] for j in range(_ODIM)]

    def group(g, carry):
        ivec = g * _L + lax.iota(jnp.int32, _L)
        igv = ig_v[pl.ds(g * _L, _L)] * _ODIM
        ggv = gg_v[pl.ds(g * _L, _L)] * _ODIM
        es = [
            jnp.maximum(
                plsc.load_gather(
                    rows_v, [ivec, jnp.full((_L,), k, jnp.int32)]),
                0.0)
            for k in range(_D)
        ]
        for j in range(_ODIM):
            acc = (plsc.load_gather(t1_v, [igv + j])
                   + plsc.load_gather(t2_v, [ggv + j]))
            for k in range(_D):
                acc = acc + es[k] * ws[j][k]
            plsc.store_scatter(
                out_v, [ivec, jnp.full((_L,), j, jnp.int32)], acc)
        return carry

    lax.fori_loop(0, _G, group, 0)
    pltpu.sync_copy(out_v, out_hbm.at[pl.ds(base, _BPW)])


def kernel(item_ids, index_group_names, garment_group_names, emb_table, W, b):
    ids = item_ids.astype(jnp.int32).reshape(_NW, _CH, _CW)
    ig = index_group_names.astype(jnp.int32).reshape(_NW, _BPW)
    gg = garment_group_names.astype(jnp.int32).reshape(_NW, _BPW)
    w16 = W[:, :_D].reshape(-1)
    t1 = (W[:, _D:_D + _NIG].T + b[None, :]).reshape(-1)
    t2 = W[:, _D + _NIG:].T.reshape(-1)
    return _tower_sc(ids, ig, gg, emb_table, w16, t1, t2)


# speedup vs baseline: 1.5178x; 1.5178x over previous
import functools

import jax
import jax.numpy as jnp
from jax import lax
from jax.experimental import pallas as pl
from jax.experimental.pallas import tpu as pltpu
from jax.experimental.pallas import tpu_sc as plsc

_N = 1000000
_B = 16384
_D = 16
_ODIM = 10
_NIG = 10
_NGG = 21

_NC, _NS, _L = 2, 16, 16
_NW = _NC * _NS
_BPW = _B // _NW
_CW = 128
_CH = _BPW // _CW
_G = _BPW // _L

_CBI = 8192
_NBLK = pl.cdiv(_N, _CBI)
_PR = _NBLK * (_CBI // 8)


def _detile_body(x_ref, o_ref):
    y = x_ref[...].T
    o_ref[...] = jnp.concatenate(
        [y[s * (_CBI // 8):(s + 1) * (_CBI // 8)] for s in range(8)], axis=1)


_detile = pl.pallas_call(
    _detile_body,
    grid=(_NBLK,),
    in_specs=[pl.BlockSpec((_D, _CBI), lambda i: (0, i))],
    out_specs=pl.BlockSpec((_CBI // 8, 128), lambda i: (i, 0)),
    out_shape=jax.ShapeDtypeStruct((_PR, 128), jnp.float32),
)

_mesh = plsc.VectorSubcoreMesh(core_axis_name="c", subcore_axis_name="s")


@functools.partial(
    pl.kernel,
    mesh=_mesh,
    out_type=jax.ShapeDtypeStruct((_ODIM, _B), jnp.float32),
    compiler_params=pltpu.CompilerParams(
        needs_layout_passes=False, use_tc_tiling_on_sc=True),
    scratch_types=[
        pltpu.VMEM((_CH, _CW), jnp.int32),
        pltpu.VMEM((_BPW, 128), jnp.float32),
        pltpu.VMEM((_BPW,), jnp.int32),
        pltpu.VMEM((_BPW,), jnp.int32),
        pltpu.VMEM((_BPW,), jnp.int32),
        pltpu.VMEM((_ODIM * _D,), jnp.float32),
        pltpu.VMEM((_NIG * _ODIM,), jnp.float32),
        pltpu.VMEM((_NGG * _ODIM,), jnp.float32),
        pltpu.VMEM((_ODIM, _BPW), jnp.float32),
        pltpu.SemaphoreType.DMA,
    ],
)
def _tower_sc(rid_hbm, off_hbm, ig_hbm, gg_hbm, p_hbm, w16_hbm, t1_hbm,
              t2_hbm, out_hbm, idx_v, rows_v, off_v, ig_v, gg_v, w16_v, t1_v,
              t2_v, out_v, sem):
    c = lax.axis_index("c")
    s = lax.axis_index("s")
    wid = s * _NC + c
    base = wid * _BPW

    pltpu.sync_copy(rid_hbm.at[wid], idx_v)
    gathers = [
        pltpu.async_copy(p_hbm.at[idx_v.at[ci]],
                         rows_v.at[pl.ds(ci * _CW, _CW)], sem)
        for ci in range(_CH)
    ]
    pltpu.sync_copy(off_hbm.at[wid], off_v)
    pltpu.sync_copy(ig_hbm.at[wid], ig_v)
    pltpu.sync_copy(gg_hbm.at[wid], gg_v)
    pltpu.sync_copy(w16_hbm, w16_v)
    pltpu.sync_copy(t1_hbm, t1_v)
    pltpu.sync_copy(t2_hbm, t2_v)
    for g in gathers:
        g.wait()

    wrows = [w16_v[pl.ds(j * _D, _D)] for j in range(_ODIM)]
    ws = [[wrows[j][k] for k in range(_D)] for j in range(_ODIM)]

    def group(g, carry):
        ivec = g * _L + lax.iota(jnp.int32, _L)
        offs = off_v[pl.ds(g * _L, _L)]
        igv = ig_v[pl.ds(g * _L, _L)] * _ODIM
        ggv = gg_v[pl.ds(g * _L, _L)] * _ODIM
        es = [
            jnp.maximum(plsc.load_gather(rows_v, [ivec, offs + k]), 0.0)
            for k in range(_D)
        ]
        for j in range(_ODIM):
            acc = (plsc.load_gather(t1_v, [igv + j])
                   + plsc.load_gather(t2_v, [ggv + j]))
            for k in range(_D):
                acc = acc + es[k] * ws[j][k]
            out_v[j, pl.ds(g * _L, _L)] = acc
        return carry

    lax.fori_loop(0, _G, group, 0)
    pltpu.sync_copy(out_v, out_hbm.at[:, pl.ds(base, _BPW)])


def kernel(item_ids, index_group_names, garment_group_names, emb_table, W, b):
    ids = item_ids.astype(jnp.int32)
    blk, l = ids // _CBI, ids % _CBI
    rid = (blk * (_CBI // 8) + l % (_CBI // 8)).reshape(_NW, _CH, _CW)
    off = ((l // (_CBI // 8)) * _D).reshape(_NW, _BPW)
    ig = index_group_names.astype(jnp.int32).reshape(_NW, _BPW)
    gg = garment_group_names.astype(jnp.int32).reshape(_NW, _BPW)
    w16 = W[:, :_D].reshape(-1)
    t1 = (W[:, _D:_D + _NIG].T + b[None, :]).reshape(-1)
    t2 = W[:, _D + _NIG:].T.reshape(-1)
    p = _detile(emb_table.T)
    return _tower_sc(rid, off, ig, gg, p, w16, t1, t2).T


# speedup vs baseline: 2.2504x; 1.4827x over previous
import functools

import numpy as np

import jax
import jax.numpy as jnp
from jax import lax
from jax.experimental import pallas as pl
from jax.experimental.pallas import tpu as pltpu
from jax.experimental.pallas import tpu_sc as plsc

_N = 1000000
_B = 16384
_D = 16
_ODIM = 10
_NIG = 10
_NGG = 21

_NC, _NS, _L = 2, 16, 16
_NW = _NC * _NS
_BPW = _B // _NW
_CW = 128
_CH = _BPW // _CW
_G = _BPW // _L

_CBI = 8192
_NBLK = pl.cdiv(_N, _CBI)
_PR = _NBLK * (_CBI // 8)


def _detile_body(x_ref, o_ref):
    x = x_ref[...]
    row = lax.broadcasted_iota(jnp.int32, (_D, 128), 0)
    col = lax.broadcasted_iota(jnp.int32, (_D, 128), 1)
    acc = None
    for s in range(8):
        xs = x[:, s * (_CBI // 8):(s + 1) * (_CBI // 8)]
        es = (col == row + 16 * s).astype(jnp.float32)
        part = jax.lax.dot_general(
            xs, es, (((0,), (0,)), ((), ())),
            preferred_element_type=jnp.float32)
        acc = part if acc is None else acc + part
    o_ref[...] = acc


_detile = pl.pallas_call(
    _detile_body,
    grid=(_NBLK,),
    in_specs=[pl.BlockSpec((_D, _CBI), lambda i: (0, i))],
    out_specs=pl.BlockSpec((_CBI // 8, 128), lambda i: (i, 0)),
    out_shape=jax.ShapeDtypeStruct((_PR, 128), jnp.float32),
)

_mesh = plsc.VectorSubcoreMesh(core_axis_name="c", subcore_axis_name="s")


@functools.partial(
    pl.kernel,
    mesh=_mesh,
    out_type=jax.ShapeDtypeStruct((_ODIM, _B), jnp.float32),
    compiler_params=pltpu.CompilerParams(
        needs_layout_passes=False, use_tc_tiling_on_sc=True),
    scratch_types=[
        pltpu.VMEM((_CH, _CW), jnp.int32),
        pltpu.VMEM((_BPW, 128), jnp.float32),
        pltpu.VMEM((_BPW,), jnp.int32),
        pltpu.VMEM((_BPW,), jnp.int32),
        pltpu.VMEM((_BPW,), jnp.int32),
        pltpu.VMEM((_ODIM * _D,), jnp.float32),
        pltpu.VMEM((_NIG * _ODIM,), jnp.float32),
        pltpu.VMEM((_NGG * _ODIM,), jnp.float32),
        pltpu.VMEM((_ODIM, _BPW), jnp.float32),
        pltpu.SemaphoreType.DMA,
    ],
)
def _tower_sc(rid_hbm, off_hbm, ig_hbm, gg_hbm, p_hbm, w16_hbm, t1_hbm,
              t2_hbm, out_hbm, idx_v, rows_v, off_v, ig_v, gg_v, w16_v, t1_v,
              t2_v, out_v, sem):
    c = lax.axis_index("c")
    s = lax.axis_index("s")
    wid = s * _NC + c
    base = wid * _BPW

    pltpu.sync_copy(rid_hbm.at[wid], idx_v)
    gathers = [
        pltpu.async_copy(p_hbm.at[idx_v.at[ci]],
                         rows_v.at[pl.ds(ci * _CW, _CW)], sem)
        for ci in range(_CH)
    ]
    pltpu.sync_copy(off_hbm.at[wid], off_v)
    pltpu.sync_copy(ig_hbm.at[wid], ig_v)
    pltpu.sync_copy(gg_hbm.at[wid], gg_v)
    pltpu.sync_copy(w16_hbm, w16_v)
    pltpu.sync_copy(t1_hbm, t1_v)
    pltpu.sync_copy(t2_hbm, t2_v)
    for g in gathers:
        g.wait()

    wrows = [w16_v[pl.ds(j * _D, _D)] for j in range(_ODIM)]
    ws = [[wrows[j][k] for k in range(_D)] for j in range(_ODIM)]

    def group(g, carry):
        ivec = g * _L + lax.iota(jnp.int32, _L)
        offs = off_v[pl.ds(g * _L, _L)]
        igv = ig_v[pl.ds(g * _L, _L)] * _ODIM
        ggv = gg_v[pl.ds(g * _L, _L)] * _ODIM
        es = [
            jnp.maximum(plsc.load_gather(rows_v, [ivec, offs + k]), 0.0)
            for k in range(_D)
        ]
        for j in range(_ODIM):
            acc = (plsc.load_gather(t1_v, [igv + j])
                   + plsc.load_gather(t2_v, [ggv + j]))
            for k in range(_D):
                acc = acc + es[k] * ws[j][k]
            out_v[j, pl.ds(g * _L, _L)] = acc
        return carry

    lax.fori_loop(0, _G, group, 0)
    pltpu.sync_copy(out_v, out_hbm.at[:, pl.ds(base, _BPW)])


def kernel(item_ids, index_group_names, garment_group_names, emb_table, W, b):
    ids = item_ids.astype(jnp.int32)
    blk, l = ids // _CBI, ids % _CBI
    rid = (blk * (_CBI // 8) + l % (_CBI // 8)).reshape(_NW, _CH, _CW)
    off = ((l // (_CBI // 8)) * _D).reshape(_NW, _BPW)
    ig = index_group_names.astype(jnp.int32).reshape(_NW, _BPW)
    gg = garment_group_names.astype(jnp.int32).reshape(_NW, _BPW)
    w16 = W[:, :_D].reshape(-1)
    t1 = (W[:, _D:_D + _NIG].T + b[None, :]).reshape(-1)
    t2 = W[:, _D + _NIG:].T.reshape(-1)
    p = _detile(emb_table.T)
    return _tower_sc(rid, off, ig, gg, p, w16, t1, t2).T
